# Initial kernel scaffold; baseline (speedup 1.0000x reference)
import jax
import jax.numpy as jnp
from jax.experimental import pallas as pl


def kernel(x, stats, conv_Wa, conv_ba, conv_g, conv_b, conv_Wb, conv_bb, bn_g, bn_b, Wf, bf, Wc1, bc1, Wc2, bc2, Wl1, bl1, l_g, l_b, Wl2, bl2, edge_index, batch):
    raise NotImplementedError("write your pallas kernel here")


# speedup vs baseline: 6.6034x; 6.6034x over previous
import functools

import jax
import jax.numpy as jnp
from jax import lax
from jax.experimental import pallas as pl
from jax.experimental.pallas import tpu as pltpu
from jax.experimental.pallas import tpu_sc as plsc

_N = 10000
_E = 320000
_H = 128
_G = 128
_NL = 3
_NC = 2
_NS = 16
_NW = _NC * _NS
_EPW = _E // _NW
_K = 80
_NB = _EPW // _K
_NP = 10240
_RPT = _NP // _NS


def _leaky(v):
    return jnp.where(v > 0, v, 0.2 * v)


def _bnorm(v, g, b):
    mu = jnp.mean(v, axis=0, keepdims=True)
    var = jnp.mean((v - mu) ** 2, axis=0, keepdims=True)
    return g * (v - mu) / jnp.sqrt(var + 1e-5) + b


def _sc_seg_sum_body(h_hbm, src_hbm, dst_hbm, zero_hbm, out_hbm,
                     src_v, dst_v, rows_v, agg_sh, sem):
    cid = lax.axis_index("c")
    sid = lax.axis_index("s")
    wid = sid * _NC + cid
    pltpu.sync_copy(zero_hbm, agg_sh.at[pl.ds(sid * _RPT, _RPT)])
    pltpu.sync_copy(src_hbm.at[wid], src_v)
    pltpu.sync_copy(dst_hbm.at[wid], dst_v)
    plsc.subcore_barrier()

    def body(j, carry):
        pltpu.async_copy(h_hbm.at[src_v.at[j]], rows_v, sem).wait()
        pltpu.sync_copy(rows_v, agg_sh.at[dst_v.at[j]], add=True)
        return carry

    lax.fori_loop(0, _NB, body, 0, unroll=False)
    plsc.subcore_barrier()
    pltpu.sync_copy(agg_sh.at[pl.ds(sid * _RPT, _RPT)],
                    out_hbm.at[cid].at[pl.ds(sid * _RPT, _RPT)])


@functools.cache
def _sc_seg_sum():
    return pl.kernel(
        _sc_seg_sum_body,
        out_type=jax.ShapeDtypeStruct((_NC, _NP, _H), jnp.float32),
        mesh=plsc.VectorSubcoreMesh(core_axis_name="c", subcore_axis_name="s"),
        scratch_types=[
            pltpu.VMEM((_NB, _K), jnp.int32),
            pltpu.VMEM((_NB, _K), jnp.int32),
            pltpu.VMEM((_K, _H), jnp.float32),
            pltpu.VMEM_SHARED((_NP, _H), jnp.float32),
            pltpu.SemaphoreType.DMA,
        ],
    )


def _mlp_body(h_ref, agg_ref, Wa_ref, ba_ref, g_ref, b_ref, Wb_ref, bb_ref,
              out_ref):
    z = h_ref[...] + agg_ref[0, :_N] + agg_ref[1, :_N]
    z = jnp.dot(z, Wa_ref[...], preferred_element_type=jnp.float32) + ba_ref[...]
    z = _leaky(z)
    z = _bnorm(z, g_ref[...], b_ref[...])
    z = jnp.dot(z, Wb_ref[...], preferred_element_type=jnp.float32) + bb_ref[...]
    out_ref[...] = _leaky(z)


_mlp = pl.pallas_call(
    _mlp_body,
    out_shape=jax.ShapeDtypeStruct((_N, _H), jnp.float32),
)


def _head_body(h_ref, batch_ref, stats_ref, bng_ref, bnb_ref, Wf_ref, bf_ref,
               Wc1_ref, bc1_ref, Wc2_ref, bc2_ref, Wl1a_ref, Wl1b_ref,
               bl1_ref, lg_ref, lb_ref, Wl2_ref, bl2_ref, out_ref):
    seg = batch_ref[...]
    gid = lax.broadcasted_iota(jnp.int32, (_G, _N), 0)
    onehot = jnp.where(seg == gid, 1.0, 0.0)
    pooled = jnp.dot(onehot, h_ref[...], preferred_element_type=jnp.float32)
    o = _bnorm(pooled, bng_ref[...], bnb_ref[...])
    o = jnp.dot(o, Wf_ref[...], preferred_element_type=jnp.float32) + bf_ref[...]
    c = jnp.dot(stats_ref[...], Wc1_ref[...],
                preferred_element_type=jnp.float32) + bc1_ref[...]
    c = jnp.maximum(c, 0.0)
    c = jnp.dot(c, Wc2_ref[...], preferred_element_type=jnp.float32) + bc2_ref[...]
    o2 = (jnp.dot(o, Wl1a_ref[...], preferred_element_type=jnp.float32)
          + jnp.dot(c, Wl1b_ref[...], preferred_element_type=jnp.float32)
          + bl1_ref[...])
    o2 = _leaky(o2)
    o2 = _bnorm(o2, lg_ref[...], lb_ref[...])
    out_ref[...] = jnp.dot(o2, Wl2_ref[...],
                           preferred_element_type=jnp.float32) + bl2_ref[...]


_head = pl.pallas_call(
    _head_body,
    out_shape=jax.ShapeDtypeStruct((_G, _H), jnp.float32),
)


def kernel(x, stats, conv_Wa, conv_ba, conv_g, conv_b, conv_Wb, conv_bb,
           bn_g, bn_b, Wf, bf, Wc1, bc1, Wc2, bc2,
           Wl1, bl1, l_g, l_b, Wl2, bl2, edge_index, batch):
    src = edge_index[0].reshape(_NW, _NB, _K)
    dst = edge_index[1].reshape(_NW, _NB, _K)
    zero = jnp.zeros((_RPT, _H), jnp.float32)
    batch2 = batch.reshape(1, _N)

    r = lambda v: v.reshape(1, -1)
    h = x
    for i in range(_NL):
        agg = _sc_seg_sum()(h, src, dst, zero)
        h = _mlp(h, agg, conv_Wa[i], r(conv_ba[i]), r(conv_g[i]),
                 r(conv_b[i]), conv_Wb[i], r(conv_bb[i]))
    return _head(h, batch2, stats, r(bn_g), r(bn_b), Wf, r(bf),
                 Wc1, r(bc1), Wc2, r(bc2), Wl1[:_H], Wl1[_H:], r(bl1),
                 r(l_g), r(l_b), Wl2, r(bl2))


# speedup vs baseline: 9.9645x; 1.5090x over previous
import functools

import jax
import jax.numpy as jnp
from jax import lax
from jax.experimental import pallas as pl
from jax.experimental.pallas import tpu as pltpu
from jax.experimental.pallas import tpu_sc as plsc

_N = 10000
_E = 320000
_H = 128
_G = 128
_NL = 3
_NC = 2
_NS = 16
_NW = _NC * _NS
_EPW = _E // _NW
_K = 80
_NB = _EPW // _K
_NCH = 5
_CH = _NB // _NCH
_NP = 10240
_RPT = _NP // _NS


def _leaky(v):
    return jnp.where(v > 0, v, 0.2 * v)


def _bnorm(v, g, b):
    mu = jnp.mean(v, axis=0, keepdims=True)
    var = jnp.mean((v - mu) ** 2, axis=0, keepdims=True)
    return g * (v - mu) / jnp.sqrt(var + 1e-5) + b


def _sc_seg_sum_body(h_hbm, src_hbm, dst_hbm, zero_hbm, out_hbm,
                     src_v, dst_v, rows0_v, rows1_v, agg_sh, sem0, sem1):
    cid = lax.axis_index("c")
    sid = lax.axis_index("s")
    wid = sid * _NC + cid
    pltpu.sync_copy(zero_hbm, agg_sh.at[pl.ds(sid * _RPT, _RPT)])
    plsc.subcore_barrier()

    def chunk_body(ch, carry):
        pltpu.sync_copy(src_hbm.at[wid * _NCH + ch], src_v)
        pltpu.sync_copy(dst_hbm.at[wid * _NCH + ch], dst_v)
        pltpu.async_copy(h_hbm.at[src_v.at[0]], rows0_v, sem0)

        def body(t, c2):
            j0 = 2 * t
            j1 = j0 + 1
            @pl.when(j1 < _CH)
            def _():
                pltpu.async_copy(h_hbm.at[src_v.at[j1]], rows1_v, sem1)
            pltpu.make_async_copy(h_hbm.at[src_v.at[j0]], rows0_v, sem0).wait()
            pltpu.sync_copy(rows0_v, agg_sh.at[dst_v.at[j0]], add=True)

            @pl.when(j0 + 2 < _CH)
            def _():
                pltpu.async_copy(h_hbm.at[src_v.at[j0 + 2]], rows0_v, sem0)

            @pl.when(j1 < _CH)
            def _():
                pltpu.make_async_copy(h_hbm.at[src_v.at[j1]], rows1_v, sem1).wait()
                pltpu.sync_copy(rows1_v, agg_sh.at[dst_v.at[j1]], add=True)
            return c2

        lax.fori_loop(0, (_CH + 1) // 2, body, 0, unroll=False)
        return carry

    lax.fori_loop(0, _NCH, chunk_body, 0, unroll=False)
    plsc.subcore_barrier()
    pltpu.sync_copy(agg_sh.at[pl.ds(sid * _RPT, _RPT)],
                    out_hbm.at[cid].at[pl.ds(sid * _RPT, _RPT)])


@functools.cache
def _sc_seg_sum():
    return pl.kernel(
        _sc_seg_sum_body,
        out_type=jax.ShapeDtypeStruct((_NC, _NP, _H), jnp.float32),
        mesh=plsc.VectorSubcoreMesh(core_axis_name="c", subcore_axis_name="s"),
        scratch_types=[
            pltpu.VMEM((_CH, _K), jnp.int32),
            pltpu.VMEM((_CH, _K), jnp.int32),
            pltpu.VMEM((_K, _H), jnp.float32),
            pltpu.VMEM((_K, _H), jnp.float32),
            pltpu.VMEM_SHARED((_NP, _H), jnp.float32),
            pltpu.SemaphoreType.DMA,
            pltpu.SemaphoreType.DMA,
        ],
    )


def _mlp_body(h_ref, agg_ref, Wa_ref, ba_ref, g_ref, b_ref, Wb_ref, bb_ref,
              out_ref):
    z = h_ref[...] + agg_ref[0, :_N] + agg_ref[1, :_N]
    z = jnp.dot(z, Wa_ref[...], preferred_element_type=jnp.float32) + ba_ref[...]
    z = _leaky(z)
    z = _bnorm(z, g_ref[...], b_ref[...])
    z = jnp.dot(z, Wb_ref[...], preferred_element_type=jnp.float32) + bb_ref[...]
    out_ref[...] = _leaky(z)


_mlp = pl.pallas_call(
    _mlp_body,
    out_shape=jax.ShapeDtypeStruct((_N, _H), jnp.float32),
)


def _head_body(h_ref, batch_ref, stats_ref, bng_ref, bnb_ref, Wf_ref, bf_ref,
               Wc1_ref, bc1_ref, Wc2_ref, bc2_ref, Wl1a_ref, Wl1b_ref,
               bl1_ref, lg_ref, lb_ref, Wl2_ref, bl2_ref, out_ref):
    seg = batch_ref[...]
    gid = lax.broadcasted_iota(jnp.int32, (_G, _N), 0)
    onehot = jnp.where(seg == gid, 1.0, 0.0)
    pooled = jnp.dot(onehot, h_ref[...], preferred_element_type=jnp.float32)
    o = _bnorm(pooled, bng_ref[...], bnb_ref[...])
    o = jnp.dot(o, Wf_ref[...], preferred_element_type=jnp.float32) + bf_ref[...]
    c = jnp.dot(stats_ref[...], Wc1_ref[...],
                preferred_element_type=jnp.float32) + bc1_ref[...]
    c = jnp.maximum(c, 0.0)
    c = jnp.dot(c, Wc2_ref[...], preferred_element_type=jnp.float32) + bc2_ref[...]
    o2 = (jnp.dot(o, Wl1a_ref[...], preferred_element_type=jnp.float32)
          + jnp.dot(c, Wl1b_ref[...], preferred_element_type=jnp.float32)
          + bl1_ref[...])
    o2 = _leaky(o2)
    o2 = _bnorm(o2, lg_ref[...], lb_ref[...])
    out_ref[...] = jnp.dot(o2, Wl2_ref[...],
                           preferred_element_type=jnp.float32) + bl2_ref[...]


_head = pl.pallas_call(
    _head_body,
    out_shape=jax.ShapeDtypeStruct((_G, _H), jnp.float32),
)


def kernel(x, stats, conv_Wa, conv_ba, conv_g, conv_b, conv_Wb, conv_bb,
           bn_g, bn_b, Wf, bf, Wc1, bc1, Wc2, bc2,
           Wl1, bl1, l_g, l_b, Wl2, bl2, edge_index, batch):
    src = edge_index[0].reshape(_NW * _NCH, _CH, _K)
    dst = edge_index[1].reshape(_NW * _NCH, _CH, _K)
    zero = jnp.zeros((_RPT, _H), jnp.float32)
    batch2 = batch.reshape(1, _N)

    r = lambda v: v.reshape(1, -1)
    h = x
    for i in range(_NL):
        agg = _sc_seg_sum()(h, src, dst, zero)
        h = _mlp(h, agg, conv_Wa[i], r(conv_ba[i]), r(conv_g[i]),
                 r(conv_b[i]), conv_Wb[i], r(conv_bb[i]))
    return _head(h, batch2, stats, r(bn_g), r(bn_b), Wf, r(bf),
                 Wc1, r(bc1), Wc2, r(bc2), Wl1[:_H], Wl1[_H:], r(bl1),
                 r(l_g), r(l_b), Wl2, r(bl2))


# speedup vs baseline: 10.7425x; 1.0781x over previous
import functools

import jax
import jax.numpy as jnp
from jax import lax
from jax.experimental import pallas as pl
from jax.experimental.pallas import tpu as pltpu
from jax.experimental.pallas import tpu_sc as plsc

_N = 10000
_E = 320000
_H = 128
_G = 128
_NL = 3
_NC = 2
_NS = 16
_NW = _NC * _NS
_EPW = _E // _NW
_K = 80
_NB = _EPW // _K
_NCH = 5
_CH = _NB // _NCH
_NP = 10240
_RPT = _NP // _NS
_CHP = _CH // 2


def _leaky(v):
    return jnp.where(v > 0, v, 0.2 * v)


def _bnorm(v, g, b):
    mu = jnp.mean(v, axis=0, keepdims=True)
    var = jnp.mean((v - mu) ** 2, axis=0, keepdims=True)
    return g * (v - mu) / jnp.sqrt(var + 1e-5) + b


def _sc_seg_sum_body(h_hbm, src_hbm, dst_hbm, zero_hbm, out_hbm,
                     src_v0, dst_v0, src_v1, dst_v1, rows0_v, rows1_v,
                     agg_sh, sem0, sem1, sem_z, sem_is, sem_id):
    cid = lax.axis_index("c")
    sid = lax.axis_index("s")
    wid = sid * _NC + cid
    zslice = agg_sh.at[pl.ds(sid * _RPT, _RPT)]
    srcb = (src_v0, src_v1)
    dstb = (dst_v0, dst_v1)
    rows = (rows0_v, rows1_v)
    sems = (sem0, sem1)

    pltpu.async_copy(zero_hbm, zslice, sem_z)
    pltpu.sync_copy(src_hbm.at[wid * _NCH], src_v0)
    pltpu.sync_copy(dst_hbm.at[wid * _NCH], dst_v0)
    pltpu.async_copy(h_hbm.at[src_v0.at[0]], rows0_v, sem0)
    pltpu.async_copy(h_hbm.at[src_v0.at[1]], rows1_v, sem1)
    pltpu.make_async_copy(zero_hbm, zslice, sem_z).wait()
    plsc.subcore_barrier()

    for ch in range(_NCH):
        q = ch % 2
        srcv, dstv = srcb[q], dstb[q]
        srcn, dstn = srcb[1 - q], dstb[1 - q]
        rA, sA = rows[q], sems[q]
        rB, sB = rows[1 - q], sems[1 - q]
        last = ch + 1 == _NCH
        if not last:
            pltpu.async_copy(src_hbm.at[wid * _NCH + ch + 1], srcn, sem_is)
            pltpu.async_copy(dst_hbm.at[wid * _NCH + ch + 1], dstn, sem_id)

        def body(t, c2, srcv=srcv, dstv=dstv, srcn=srcn, dstn=dstn,
                 rA=rA, sA=sA, rB=rB, sB=sB, last=last, ch=ch):
            j0 = 2 * t
            j1 = j0 + 1
            pltpu.make_async_copy(h_hbm.at[srcv.at[j0]], rA, sA).wait()
            pltpu.sync_copy(rA, agg_sh.at[dstv.at[j0]], add=True)
            pltpu.async_copy(h_hbm.at[srcv.at[j0 + 2]], rA, sA)
            pltpu.make_async_copy(h_hbm.at[srcv.at[j1]], rB, sB).wait()
            pltpu.sync_copy(rB, agg_sh.at[dstv.at[j1]], add=True)

            @pl.when(t < _CHP - 1)
            def _():
                pltpu.async_copy(h_hbm.at[srcv.at[j1 + 2]], rB, sB)

            if not last:
                @pl.when(t == _CHP - 1)
                def _():
                    nxt = wid * _NCH + ch + 1
                    pltpu.make_async_copy(src_hbm.at[nxt], srcn, sem_is).wait()
                    pltpu.make_async_copy(dst_hbm.at[nxt], dstn, sem_id).wait()
                    pltpu.async_copy(h_hbm.at[srcn.at[0]], rB, sB)
            return c2

        lax.fori_loop(0, _CHP, body, 0, unroll=False)
        jl = _CH - 1
        pltpu.make_async_copy(h_hbm.at[srcv.at[jl]], rA, sA).wait()
        pltpu.sync_copy(rA, agg_sh.at[dstv.at[jl]], add=True)
        if not last:
            pltpu.async_copy(h_hbm.at[srcn.at[1]], rA, sA)

    plsc.subcore_barrier()
    pltpu.sync_copy(agg_sh.at[pl.ds(sid * _RPT, _RPT)],
                    out_hbm.at[cid].at[pl.ds(sid * _RPT, _RPT)])


@functools.cache
def _sc_seg_sum():
    return pl.kernel(
        _sc_seg_sum_body,
        out_type=jax.ShapeDtypeStruct((_NC, _NP, _H), jnp.float32),
        mesh=plsc.VectorSubcoreMesh(core_axis_name="c", subcore_axis_name="s"),
        scratch_types=[
            pltpu.VMEM((_CH, _K), jnp.int32),
            pltpu.VMEM((_CH, _K), jnp.int32),
            pltpu.VMEM((_CH, _K), jnp.int32),
            pltpu.VMEM((_CH, _K), jnp.int32),
            pltpu.VMEM((_K, _H), jnp.float32),
            pltpu.VMEM((_K, _H), jnp.float32),
            pltpu.VMEM_SHARED((_NP, _H), jnp.float32),
            pltpu.SemaphoreType.DMA,
            pltpu.SemaphoreType.DMA,
            pltpu.SemaphoreType.DMA,
            pltpu.SemaphoreType.DMA,
            pltpu.SemaphoreType.DMA,
        ],
    )


def _mlp_body(h_ref, agg_ref, Wa_ref, ba_ref, g_ref, b_ref, Wb_ref, bb_ref,
              out_ref):
    z = h_ref[...] + agg_ref[0, :_N] + agg_ref[1, :_N]
    z = jnp.dot(z, Wa_ref[...], preferred_element_type=jnp.float32) + ba_ref[...]
    z = _leaky(z)
    z = _bnorm(z, g_ref[...], b_ref[...])
    z = jnp.dot(z, Wb_ref[...], preferred_element_type=jnp.float32) + bb_ref[...]
    out_ref[...] = _leaky(z)


_mlp = pl.pallas_call(
    _mlp_body,
    out_shape=jax.ShapeDtypeStruct((_N, _H), jnp.float32),
)


def _head_body(h_ref, batch_ref, stats_ref, bng_ref, bnb_ref, Wf_ref, bf_ref,
               Wc1_ref, bc1_ref, Wc2_ref, bc2_ref, Wl1a_ref, Wl1b_ref,
               bl1_ref, lg_ref, lb_ref, Wl2_ref, bl2_ref, out_ref):
    seg = batch_ref[...]
    gid = lax.broadcasted_iota(jnp.int32, (_G, _N), 0)
    onehot = jnp.where(seg == gid, 1.0, 0.0)
    pooled = jnp.dot(onehot, h_ref[...], preferred_element_type=jnp.float32)
    o = _bnorm(pooled, bng_ref[...], bnb_ref[...])
    o = jnp.dot(o, Wf_ref[...], preferred_element_type=jnp.float32) + bf_ref[...]
    c = jnp.dot(stats_ref[...], Wc1_ref[...],
                preferred_element_type=jnp.float32) + bc1_ref[...]
    c = jnp.maximum(c, 0.0)
    c = jnp.dot(c, Wc2_ref[...], preferred_element_type=jnp.float32) + bc2_ref[...]
    o2 = (jnp.dot(o, Wl1a_ref[...], preferred_element_type=jnp.float32)
          + jnp.dot(c, Wl1b_ref[...], preferred_element_type=jnp.float32)
          + bl1_ref[...])
    o2 = _leaky(o2)
    o2 = _bnorm(o2, lg_ref[...], lb_ref[...])
    out_ref[...] = jnp.dot(o2, Wl2_ref[...],
                           preferred_element_type=jnp.float32) + bl2_ref[...]


_head = pl.pallas_call(
    _head_body,
    out_shape=jax.ShapeDtypeStruct((_G, _H), jnp.float32),
)


def kernel(x, stats, conv_Wa, conv_ba, conv_g, conv_b, conv_Wb, conv_bb,
           bn_g, bn_b, Wf, bf, Wc1, bc1, Wc2, bc2,
           Wl1, bl1, l_g, l_b, Wl2, bl2, edge_index, batch):
    src = edge_index[0].reshape(_NW * _NCH, _CH, _K)
    dst = edge_index[1].reshape(_NW * _NCH, _CH, _K)
    zero = jnp.zeros((_RPT, _H), jnp.float32)
    batch2 = batch.reshape(1, _N)

    r = lambda v: v.reshape(1, -1)
    h = x
    for i in range(_NL):
        agg = _sc_seg_sum()(h, src, dst, zero)
        h = _mlp(h, agg, conv_Wa[i], r(conv_ba[i]), r(conv_g[i]),
                 r(conv_b[i]), conv_Wb[i], r(conv_bb[i]))
    return _head(h, batch2, stats, r(bn_g), r(bn_b), Wf, r(bf),
                 Wc1, r(bc1), Wc2, r(bc2), Wl1[:_H], Wl1[_H:], r(bl1),
                 r(l_g), r(l_b), Wl2, r(bl2))
